# baseline (speedup 1.0000x reference)
import functools

import jax
import jax.numpy as jnp
from jax import lax
from jax.experimental import pallas as pl
from jax.experimental.pallas import tpu as pltpu
from jax.experimental.pallas import tpu_sc as plsc

VOCAB = 1000000
EMBED_DIM = 32
BATCH = 16384
HIST = 200

NC, NS = 2, 16
NW = NC * NS
IDXW = 128
BPW = BATCH // NW
IPH = BPW // IDXW
LANES = 16


def _body(table_hbm, idx_hbm, out_hbm,
          ibuf0, ibuf1, gbuf0, gbuf1, tbuf0, tbuf1,
          si0, si1, sg0, sg1, so0, so1):
    wid = lax.axis_index("s") * NC + lax.axis_index("c")
    ibufs = (ibuf0, ibuf1)
    gbufs = (gbuf0, gbuf1)
    tbufs = (tbuf0, tbuf1)
    sis = (si0, si1)
    sgs = (sg0, sg1)
    sos = (so0, so1)

    irow0 = wid * IPH
    col0 = wid * BPW

    def start_idx_load(h, b):
        pltpu.async_copy(
            idx_hbm.at[pl.ds(h, 1), pl.ds(irow0, IPH), :], ibufs[b], sis[b])

    def wait_idx(b):
        pltpu.make_async_copy(
            idx_hbm.at[pl.ds(0, 1), pl.ds(0, IPH), :], ibufs[b], sis[b]).wait()

    def start_gathers(b):
        for j in range(IPH):
            pltpu.async_copy(
                table_hbm.at[ibufs[b].at[0, j]],
                gbufs[b].at[pl.ds(j * IDXW, IDXW), :],
                sgs[b])

    def wait_gathers(b):
        for j in range(IPH):
            pltpu.make_async_copy(
                table_hbm.at[ibufs[b].at[0, j]],
                gbufs[b].at[pl.ds(j * IDXW, IDXW), :],
                sgs[b]).wait()

    def transpose(b):
        gbuf, tbuf = gbufs[b], tbufs[b]
        lanes = lax.iota(jnp.int32, LANES)

        def tcol(c, carry):
            c0 = c * LANES
            rows = c0 + lanes
            for j in range(EMBED_DIM):
                cols = jnp.full((LANES,), j, jnp.int32)
                vals = plsc.load_gather(gbuf, [rows, cols])
                tbuf[0, j, pl.ds(c0, LANES)] = vals
            return carry

        lax.fori_loop(0, BPW // LANES, tcol, 0)

    def start_store(h, b):
        pltpu.async_copy(
            tbufs[b], out_hbm.at[pl.ds(h, 1), :, pl.ds(col0, BPW)], sos[b])

    def wait_store(b):
        pltpu.make_async_copy(
            tbufs[b], out_hbm.at[pl.ds(0, 1), :, pl.ds(col0, BPW)],
            sos[b]).wait()

    def half_step(h, b):
        wait_gathers(b)

        @pl.when(h + 1 < HIST)
        def _():
            wait_idx(1 - b)
            start_gathers(1 - b)

        @pl.when(h + 2 < HIST)
        def _():
            start_idx_load(h + 2, b)

        @pl.when(h >= 2)
        def _():
            wait_store(b)

        transpose(b)
        start_store(h, b)

    start_idx_load(0, 0)
    wait_idx(0)
    start_gathers(0)
    start_idx_load(1, 1)

    def loop_body(t, carry):
        half_step(2 * t, 0)
        half_step(2 * t + 1, 1)
        return carry

    lax.fori_loop(0, HIST // 2, loop_body, 0)

    wait_store(0)
    wait_store(1)


@jax.jit
def _lookup(idx3d, table):
    mesh = plsc.VectorSubcoreMesh(
        core_axis_name="c", subcore_axis_name="s",
        num_cores=NC, num_subcores=NS)
    f = pl.kernel(
        _body,
        out_type=jax.ShapeDtypeStruct((HIST, EMBED_DIM, BATCH), jnp.float32),
        mesh=mesh,
        compiler_params=pltpu.CompilerParams(
            use_tc_tiling_on_sc=False, needs_layout_passes=False),
        scratch_types=[
            pltpu.VMEM((1, IPH, IDXW), jnp.int32),
            pltpu.VMEM((1, IPH, IDXW), jnp.int32),
            pltpu.VMEM((BPW, EMBED_DIM), jnp.float32),
            pltpu.VMEM((BPW, EMBED_DIM), jnp.float32),
            pltpu.VMEM((1, EMBED_DIM, BPW), jnp.float32),
            pltpu.VMEM((1, EMBED_DIM, BPW), jnp.float32),
            pltpu.SemaphoreType.DMA,
            pltpu.SemaphoreType.DMA,
            pltpu.SemaphoreType.DMA,
            pltpu.SemaphoreType.DMA,
            pltpu.SemaphoreType.DMA,
            pltpu.SemaphoreType.DMA,
        ],
    )
    return f(table, idx3d)


def kernel(x, table):
    idx3d = x.astype(jnp.int32).T.reshape(HIST, BATCH // IDXW, IDXW)
    out = _lookup(idx3d, table)
    return jnp.transpose(out, (2, 0, 1))


# speedup vs baseline: 1.3687x; 1.3687x over previous
import functools

import jax
import jax.numpy as jnp
from jax import lax
from jax.experimental import pallas as pl
from jax.experimental.pallas import tpu as pltpu
from jax.experimental.pallas import tpu_sc as plsc

VOCAB = 1000000
EMBED_DIM = 32
BATCH = 16384
HIST = 200

NC, NS = 2, 16
NW = NC * NS
IDXW = 128
BPW = BATCH // NW
IPH = BPW // IDXW
LANES = 16
JB = EMBED_DIM // 8
TILE_F32 = 8 * 128
SLAB = BPW * EMBED_DIM
OUT_FLAT = HIST * EMBED_DIM * BATCH


def _body(table_hbm, idx_hbm, out_hbm,
          ibuf0, ibuf1, gbuf0, gbuf1, tbuf0, tbuf1,
          si0, si1, sg0, sg1, so0, so1):
    wid = lax.axis_index("s") * NC + lax.axis_index("c")
    ibufs = (ibuf0, ibuf1)
    gbufs = (gbuf0, gbuf1)
    tbufs = (tbuf0, tbuf1)
    sis = (si0, si1)
    sgs = (sg0, sg1)
    sos = (so0, so1)

    irow0 = wid * IPH

    def start_idx_load(h, b):
        pltpu.async_copy(
            idx_hbm.at[pl.ds(h, 1), pl.ds(irow0, IPH), :], ibufs[b], sis[b])

    def wait_idx(b):
        pltpu.make_async_copy(
            idx_hbm.at[pl.ds(0, 1), pl.ds(0, IPH), :], ibufs[b], sis[b]).wait()

    def start_gathers(b):
        for j in range(IPH):
            pltpu.async_copy(
                table_hbm.at[ibufs[b].at[0, j]],
                gbufs[b].at[pl.ds(j * IDXW, IDXW), :],
                sgs[b])

    def wait_gathers(b):
        for j in range(IPH):
            pltpu.make_async_copy(
                table_hbm.at[ibufs[b].at[0, j]],
                gbufs[b].at[pl.ds(j * IDXW, IDXW), :],
                sgs[b]).wait()

    def transpose(b):
        gbuf, tbuf = gbufs[b], tbufs[b]
        lanes = lax.iota(jnp.int32, LANES)

        for jb in range(JB):
            def tcol(m, carry, jb=jb):
                ib = m // 8
                r = m - ib * 8
                rows0 = ib * IDXW + lanes
                cols = jnp.full((LANES,), jb * 8, jnp.int32) + r
                base = jb * (JB * TILE_F32) + m * 128
                vals = [
                    plsc.load_gather(gbuf, [rows0 + (k * LANES), cols])
                    for k in range(IDXW // LANES)
                ]
                for k in range(IDXW // LANES):
                    tbuf[pl.ds(base + k * LANES, LANES)] = vals[k]
                return carry

            lax.fori_loop(0, JB * 8, tcol, 0)

    def start_store(h, b):
        for jb in range(JB):
            off = h * (JB * 128 * TILE_F32) + jb * (128 * TILE_F32) \
                + wid * (JB * TILE_F32)
            pltpu.async_copy(
                tbufs[b].at[pl.ds(jb * (JB * TILE_F32), JB * TILE_F32)],
                out_hbm.at[pl.ds(off, JB * TILE_F32)],
                sos[b])

    def wait_store(b):
        for jb in range(JB):
            pltpu.make_async_copy(
                tbufs[b].at[pl.ds(jb * (JB * TILE_F32), JB * TILE_F32)],
                out_hbm.at[pl.ds(0, JB * TILE_F32)],
                sos[b]).wait()

    def half_step(h, b):
        wait_gathers(b)

        @pl.when(h + 1 < HIST)
        def _():
            wait_idx(1 - b)
            start_gathers(1 - b)

        @pl.when(h + 2 < HIST)
        def _():
            start_idx_load(h + 2, b)

        @pl.when(h >= 2)
        def _():
            wait_store(b)

        transpose(b)
        start_store(h, b)

    start_idx_load(0, 0)
    wait_idx(0)
    start_gathers(0)
    start_idx_load(1, 1)

    def loop_body(t, carry):
        half_step(2 * t, 0)
        half_step(2 * t + 1, 1)
        return carry

    lax.fori_loop(0, HIST // 2, loop_body, 0)

    wait_store(0)
    wait_store(1)


@jax.jit
def _lookup(idx3d, table):
    mesh = plsc.VectorSubcoreMesh(
        core_axis_name="c", subcore_axis_name="s",
        num_cores=NC, num_subcores=NS)
    f = pl.kernel(
        _body,
        out_type=jax.ShapeDtypeStruct((OUT_FLAT,), jnp.float32),
        mesh=mesh,
        compiler_params=pltpu.CompilerParams(
            use_tc_tiling_on_sc=False, needs_layout_passes=False),
        scratch_types=[
            pltpu.VMEM((1, IPH, IDXW), jnp.int32),
            pltpu.VMEM((1, IPH, IDXW), jnp.int32),
            pltpu.VMEM((BPW, EMBED_DIM), jnp.float32),
            pltpu.VMEM((BPW, EMBED_DIM), jnp.float32),
            pltpu.VMEM((SLAB,), jnp.float32),
            pltpu.VMEM((SLAB,), jnp.float32),
            pltpu.SemaphoreType.DMA,
            pltpu.SemaphoreType.DMA,
            pltpu.SemaphoreType.DMA,
            pltpu.SemaphoreType.DMA,
            pltpu.SemaphoreType.DMA,
            pltpu.SemaphoreType.DMA,
        ],
    )
    return f(table, idx3d)


def kernel(x, table):
    idx3d = x.astype(jnp.int32).T.reshape(HIST, BATCH // IDXW, IDXW)
    flat = _lookup(idx3d, table)
    y = flat.reshape(HIST, JB, BATCH // IDXW, 8, IDXW)
    y = y.transpose(0, 1, 3, 2, 4).reshape(HIST, EMBED_DIM, BATCH)
    return jnp.transpose(y, (2, 0, 1))


# speedup vs baseline: 2.5852x; 1.8888x over previous
import functools

import jax
import jax.numpy as jnp
from jax import lax
from jax.experimental import pallas as pl
from jax.experimental.pallas import tpu as pltpu
from jax.experimental.pallas import tpu_sc as plsc

VOCAB = 1000000
EMBED_DIM = 32
BATCH = 16384
HIST = 200

NC, NS = 2, 16
NW = NC * NS
IDXW = 128
BPW = BATCH // NW
IPH = BPW // IDXW
LANES = 16
JB = EMBED_DIM // 8
TILE_F32 = 8 * 128
SLAB = BPW * EMBED_DIM
OUT_FLAT = HIST * EMBED_DIM * BATCH


def _body(table_hbm, idx_hbm, out_hbm,
          ibuf0, ibuf1, gbuf0, gbuf1, tbuf0, tbuf1,
          si0, si1, sg0, sg1, so0, so1):
    wid = lax.axis_index("s") * NC + lax.axis_index("c")
    ibufs = (ibuf0, ibuf1)
    gbufs = (gbuf0, gbuf1)
    tbufs = (tbuf0, tbuf1)
    sis = (si0, si1)
    sgs = (sg0, sg1)
    sos = (so0, so1)

    irow0 = wid * IPH

    def start_idx_load(h, b):
        pltpu.async_copy(
            idx_hbm.at[pl.ds(h, 1), pl.ds(irow0, IPH), :], ibufs[b], sis[b])

    def wait_idx(b):
        pltpu.make_async_copy(
            idx_hbm.at[pl.ds(0, 1), pl.ds(0, IPH), :], ibufs[b], sis[b]).wait()

    def start_gathers(b):
        for j in range(IPH):
            pltpu.async_copy(
                table_hbm.at[ibufs[b].at[0, j]],
                gbufs[b].at[pl.ds(j * IDXW, IDXW), :],
                sgs[b])

    def wait_gathers(b):
        for j in range(IPH):
            pltpu.make_async_copy(
                table_hbm.at[ibufs[b].at[0, j]],
                gbufs[b].at[pl.ds(j * IDXW, IDXW), :],
                sgs[b]).wait()

    def transpose(b):
        gbuf, tbuf = gbufs[b], tbufs[b]
        lanes = lax.iota(jnp.int32, LANES)

        def trow(r0, carry):
            j = (r0 + lanes) & 31
            dstc = ((j >> 3) << 12) + ((j & 7) << 7) + lanes
            for ib in range(IPH):
                for c0 in range(0, IDXW, LANES):
                    rows = lanes + (ib * IDXW + c0)
                    vals = plsc.load_gather(gbuf, [rows, j])
                    plsc.store_scatter(
                        tbuf, [dstc + (ib * 1024 + c0)], vals)
            return carry

        lax.fori_loop(0, EMBED_DIM, trow, 0)

    def start_store(h, b):
        for jb in range(JB):
            off = h * (JB * 128 * TILE_F32) + jb * (128 * TILE_F32) \
                + wid * (JB * TILE_F32)
            pltpu.async_copy(
                tbufs[b].at[pl.ds(jb * (JB * TILE_F32), JB * TILE_F32)],
                out_hbm.at[pl.ds(off, JB * TILE_F32)],
                sos[b])

    def wait_store(b):
        for jb in range(JB):
            pltpu.make_async_copy(
                tbufs[b].at[pl.ds(jb * (JB * TILE_F32), JB * TILE_F32)],
                out_hbm.at[pl.ds(0, JB * TILE_F32)],
                sos[b]).wait()

    def half_step(h, b):
        wait_gathers(b)

        @pl.when(h + 1 < HIST)
        def _():
            wait_idx(1 - b)
            start_gathers(1 - b)

        @pl.when(h + 2 < HIST)
        def _():
            start_idx_load(h + 2, b)

        @pl.when(h >= 2)
        def _():
            wait_store(b)

        transpose(b)
        start_store(h, b)

    start_idx_load(0, 0)
    wait_idx(0)
    start_gathers(0)
    start_idx_load(1, 1)

    def loop_body(t, carry):
        half_step(2 * t, 0)
        half_step(2 * t + 1, 1)
        return carry

    lax.fori_loop(0, HIST // 2, loop_body, 0)

    wait_store(0)
    wait_store(1)


@jax.jit
def _lookup(idx3d, table):
    mesh = plsc.VectorSubcoreMesh(
        core_axis_name="c", subcore_axis_name="s",
        num_cores=NC, num_subcores=NS)
    f = pl.kernel(
        _body,
        out_type=jax.ShapeDtypeStruct((OUT_FLAT,), jnp.float32),
        mesh=mesh,
        compiler_params=pltpu.CompilerParams(
            use_tc_tiling_on_sc=False, needs_layout_passes=False),
        scratch_types=[
            pltpu.VMEM((1, IPH, IDXW), jnp.int32),
            pltpu.VMEM((1, IPH, IDXW), jnp.int32),
            pltpu.VMEM((BPW, EMBED_DIM), jnp.float32),
            pltpu.VMEM((BPW, EMBED_DIM), jnp.float32),
            pltpu.VMEM((SLAB,), jnp.float32),
            pltpu.VMEM((SLAB,), jnp.float32),
            pltpu.SemaphoreType.DMA,
            pltpu.SemaphoreType.DMA,
            pltpu.SemaphoreType.DMA,
            pltpu.SemaphoreType.DMA,
            pltpu.SemaphoreType.DMA,
            pltpu.SemaphoreType.DMA,
        ],
    )
    return f(table, idx3d)


def kernel(x, table):
    idx3d = x.astype(jnp.int32).T.reshape(HIST, BATCH // IDXW, IDXW)
    flat = _lookup(idx3d, table)
    y = flat.reshape(HIST, JB, BATCH // IDXW, 8, IDXW)
    y = y.transpose(0, 1, 3, 2, 4).reshape(HIST, EMBED_DIM, BATCH)
    return jnp.transpose(y, (2, 0, 1))
